# baseline (speedup 1.0000x reference)
import dataclasses
import functools

import jax
import jax.numpy as jnp
from jax import lax
from jax.experimental import pallas as pl
from jax.experimental.pallas import tpu as pltpu
from jax.experimental.pallas import tpu_sc as plsc

_VPAD = 128
_NC, _NS = 2, 16
_NW = _NC * _NS
_LANES = 16
_CHUNK = 256
_CSL = 56


@functools.partial(jax.jit, static_argnames=("ncol",))
def _sc_call(idx, scores, w1, w2, prior, bias, thresh, *, ncol):
    nrow = idx.shape[0]
    rows_w = nrow // _NW
    nch = rows_w // _CHUNK
    mesh = plsc.VectorSubcoreMesh(core_axis_name="c", subcore_axis_name="s")
    cp = pltpu.CompilerParams()
    if "needs_layout_passes" in pltpu.CompilerParams.__dataclass_fields__:
        cp = dataclasses.replace(cp, needs_layout_passes=False)
    cp = dataclasses.replace(cp, use_tc_tiling_on_sc=False)

    @functools.partial(
        pl.kernel,
        out_type=[
            jax.ShapeDtypeStruct((nrow, 128), jnp.float32),
            jax.ShapeDtypeStruct((nrow, 128), jnp.int32),
        ],
        mesh=mesh,
        scratch_types=[
            pltpu.VMEM((_CHUNK, _CSL), jnp.int32),
            pltpu.VMEM((_CHUNK, _CSL), jnp.float32),
            pltpu.VMEM((_CHUNK, _CSL), jnp.float32),
            pltpu.VMEM((_CHUNK, _CSL), jnp.int32),
            pltpu.VMEM((_VPAD,), jnp.float32),
            pltpu.VMEM((_VPAD,), jnp.float32),
            pltpu.VMEM((_VPAD,), jnp.float32),
            pltpu.VMEM((_VPAD,), jnp.float32),
            pltpu.VMEM((_VPAD,), jnp.float32),
        ],
        compiler_params=cp,
    )
    def body(idx_hbm, s_hbm, w1_hbm, w2_hbm, pr_hbm, bi_hbm, th_hbm,
             cal_hbm, mask_hbm,
             idx_v, s_v, cal_v, m_v, w1_v, c_v, pr_v, bi_v, th_v):
        wid = lax.axis_index("s") * _NC + lax.axis_index("c")
        row0 = wid * rows_w
        pltpu.sync_copy(w1_hbm, w1_v)
        pltpu.sync_copy(w2_hbm, c_v)
        pltpu.sync_copy(pr_hbm, pr_v)
        pltpu.sync_copy(bi_hbm, bi_v)
        pltpu.sync_copy(th_hbm, th_v)

        @pl.loop(0, _VPAD, step=_LANES)
        def _(i):
            sl = pl.ds(i, _LANES)
            c_v[sl] = -(c_v[sl] * pr_v[sl] + bi_v[sl])
            w1_v[sl] = -w1_v[sl]

        col_starts = list(range(0, ncol - _LANES, _LANES)) + [ncol - _LANES]
        csl = pl.ds(0, _CSL)

        @pl.loop(0, nch)
        def _(ch):
            rsl = pl.ds(row0 + ch * _CHUNK, _CHUNK)
            pltpu.sync_copy(idx_hbm.at[rsl, csl], idx_v)
            pltpu.sync_copy(s_hbm.at[rsl, csl], s_v)

            @plsc.parallel_loop(0, _CHUNK, unroll=4)
            def _(r):
                for c in col_starts:
                    sl = pl.ds(c, _LANES)
                    ids = idx_v[r, sl]
                    na = plsc.load_gather(w1_v, [ids])
                    nc2 = plsc.load_gather(c_v, [ids])
                    th = plsc.load_gather(th_v, [ids])
                    e = jnp.exp(na * s_v[r, sl] + nc2)
                    cal = 1.0 / (1.0 + e)
                    cal_v[r, sl] = cal
                    m_v[r, sl] = jnp.where(cal > th, jnp.int32(1), jnp.int32(0))

            pltpu.sync_copy(cal_v, cal_hbm.at[rsl, csl])
            pltpu.sync_copy(m_v, mask_hbm.at[rsl, csl])

    return body(idx, scores, w1, w2, prior, bias, thresh)


_ROWBLK = 1024


@jax.jit
def _tc_pack(type_ids, scores):
    b, l = type_ids.shape

    def body(i_ref, s_ref, oi_ref, os_ref):
        oi_ref[:, :l] = i_ref[...].astype(jnp.int32)
        os_ref[:, :l] = s_ref[...]

    return pl.pallas_call(
        body,
        grid=(b // _ROWBLK,),
        in_specs=[
            pl.BlockSpec((_ROWBLK, l), lambda i: (i, 0)),
            pl.BlockSpec((_ROWBLK, l), lambda i: (i, 0)),
        ],
        out_specs=[
            pl.BlockSpec((_ROWBLK, 128), lambda i: (i, 0)),
            pl.BlockSpec((_ROWBLK, 128), lambda i: (i, 0)),
        ],
        out_shape=[
            jax.ShapeDtypeStruct((b, 128), jnp.int32),
            jax.ShapeDtypeStruct((b, 128), jnp.float32),
        ],
        compiler_params=pltpu.CompilerParams(
            dimension_semantics=("parallel",)),
    )(type_ids, scores)


@functools.partial(jax.jit, static_argnames=("ncol",))
def _tc_unpack(cal128, m128, *, ncol):
    b = cal128.shape[0]

    def body(c_ref, m_ref, oc_ref, om_ref):
        oc_ref[...] = c_ref[:, :ncol]
        om_ref[...] = m_ref[:, :ncol] != 0

    return pl.pallas_call(
        body,
        grid=(b // _ROWBLK,),
        in_specs=[
            pl.BlockSpec((_ROWBLK, 128), lambda i: (i, 0)),
            pl.BlockSpec((_ROWBLK, 128), lambda i: (i, 0)),
        ],
        out_specs=[
            pl.BlockSpec((_ROWBLK, ncol), lambda i: (i, 0)),
            pl.BlockSpec((_ROWBLK, ncol), lambda i: (i, 0)),
        ],
        out_shape=[
            jax.ShapeDtypeStruct((b, ncol), jnp.float32),
            jax.ShapeDtypeStruct((b, ncol), jnp.bool_),
        ],
        compiler_params=pltpu.CompilerParams(
            dimension_semantics=("parallel",)),
    )(cal128, m128)


def kernel(type_ids, scores, prior, weights, bias, threshold):
    v = prior.shape[0]
    pad = _VPAD - v
    ncol = type_ids.shape[1]
    idx, s = _tc_pack(type_ids, scores)
    w1 = jnp.pad(weights[:, 0], (0, pad))
    w2 = jnp.pad(weights[:, 1], (0, pad))
    pr = jnp.pad(prior, (0, pad))
    bi = jnp.pad(bias, (0, pad))
    th = jnp.pad(threshold, (0, pad))
    cal128, m128 = _sc_call(idx, s, w1, w2, pr, bi, th, ncol=ncol)
    return _tc_unpack(cal128, m128, ncol=ncol)


# speedup vs baseline: 1.4723x; 1.4723x over previous
import dataclasses
import functools

import jax
import jax.numpy as jnp
from jax import lax
from jax.experimental import pallas as pl
from jax.experimental.pallas import tpu as pltpu
from jax.experimental.pallas import tpu_sc as plsc

_VPAD = 128
_NC, _NS = 2, 16
_NW = _NC * _NS
_LANES = 16
_CHUNK = 256
_CSL = 56


@functools.partial(jax.jit, static_argnames=("ncol",))
def _sc_call(idx, scores, tabs, *, ncol):
    nrow = idx.shape[0]
    rows_w = nrow // _NW
    nch = rows_w // _CHUNK
    assert nch % 2 == 0
    mesh = plsc.VectorSubcoreMesh(core_axis_name="c", subcore_axis_name="s")
    cp = pltpu.CompilerParams()
    if "needs_layout_passes" in pltpu.CompilerParams.__dataclass_fields__:
        cp = dataclasses.replace(cp, needs_layout_passes=False)
    cp = dataclasses.replace(cp, use_tc_tiling_on_sc=False)

    @functools.partial(
        pl.kernel,
        out_type=[
            jax.ShapeDtypeStruct((nrow, 128), jnp.float32),
            jax.ShapeDtypeStruct((nrow, 128), jnp.int32),
        ],
        mesh=mesh,
        scratch_types=[
            pltpu.VMEM((2, _CHUNK, _CSL), jnp.int32),
            pltpu.VMEM((2, _CHUNK, _CSL), jnp.float32),
            pltpu.VMEM((2, _CHUNK, _CSL), jnp.float32),
            pltpu.VMEM((2, _CHUNK, _CSL), jnp.int32),
            pltpu.VMEM((5, _VPAD), jnp.float32),
            pltpu.SemaphoreType.DMA,
            pltpu.SemaphoreType.DMA,
            pltpu.SemaphoreType.DMA,
        ],
        compiler_params=cp,
    )
    def body(idx_hbm, s_hbm, tab_hbm, cal_hbm, mask_hbm,
             idx_v, s_v, cal_v, m_v, tab_v, sem_a, sem_b, sem_o):
        wid = lax.axis_index("s") * _NC + lax.axis_index("c")
        row0 = wid * rows_w
        csl = pl.ds(0, _CSL)
        sems = (sem_a, sem_b)

        def rsl(ch):
            return pl.ds(row0 + ch * _CHUNK, _CHUNK)

        def start_in(ch, buf):
            a = pltpu.async_copy(idx_hbm.at[rsl(ch), csl], idx_v.at[buf],
                                 sems[buf])
            b = pltpu.async_copy(s_hbm.at[rsl(ch), csl], s_v.at[buf],
                                 sems[buf])
            return a, b

        in0 = start_in(0, 0)
        pltpu.sync_copy(tab_hbm, tab_v)
        in1 = start_in(1, 1)

        @pl.loop(0, _VPAD, step=_LANES)
        def _(i):
            sl = pl.ds(i, _LANES)
            tab_v[1, sl] = -(tab_v[1, sl] * tab_v[2, sl] + tab_v[3, sl])
            tab_v[0, sl] = -tab_v[0, sl]

        col_starts = list(range(0, ncol - _LANES, _LANES)) + [ncol - _LANES]

        na_t = tab_v.at[0]
        nc_t = tab_v.at[1]
        th_t = tab_v.at[4]

        def compute(buf):
            @plsc.parallel_loop(0, _CHUNK, unroll=4)
            def _(r):
                for c in col_starts:
                    sl = pl.ds(c, _LANES)
                    ids = idx_v[buf, r, sl]
                    na = plsc.load_gather(na_t, [ids])
                    nc2 = plsc.load_gather(nc_t, [ids])
                    th = plsc.load_gather(th_t, [ids])
                    e = jnp.exp(na * s_v[buf, r, sl] + nc2)
                    cal = 1.0 / (1.0 + e)
                    cal_v[buf, r, sl] = cal
                    m_v[buf, r, sl] = jnp.where(cal > th, jnp.int32(1),
                                                jnp.int32(0))

        def drain_out(descs):
            for d in descs:
                d.wait()

        prev_out = ()
        for ch in range(nch):
            buf = ch % 2
            cur_in = in0 if buf == 0 else in1
            for d in cur_in:
                d.wait()
            compute(buf)
            drain_out(prev_out)
            oc = pltpu.async_copy(cal_v.at[buf], cal_hbm.at[rsl(ch), csl],
                                  sem_o)
            om = pltpu.async_copy(m_v.at[buf], mask_hbm.at[rsl(ch), csl],
                                  sem_o)
            prev_out = (oc, om)
            if ch + 2 < nch:
                nxt = start_in(ch + 2, buf)
                if buf == 0:
                    in0 = nxt
                else:
                    in1 = nxt
        drain_out(prev_out)

    return body(idx, scores, tabs)


def kernel(type_ids, scores, prior, weights, bias, threshold):
    v = prior.shape[0]
    pad = _VPAD - v
    ncol = type_ids.shape[1]
    cpad = ((0, 0), (0, 128 - ncol))
    idx = jnp.pad(type_ids.astype(jnp.int32), cpad)
    s = jnp.pad(scores, cpad)
    tabs = jnp.pad(
        jnp.stack([weights[:, 0], weights[:, 1], prior, bias, threshold]),
        ((0, 0), (0, pad)))
    cal, mask = _sc_call(idx, s, tabs, ncol=ncol)
    return cal[:, :ncol], mask[:, :ncol].astype(jnp.bool_)
